# baseline (speedup 1.0000x reference)
import functools

import jax
import jax.numpy as jnp
from jax import lax
from jax.experimental import pallas as pl
from jax.experimental.pallas import tpu as pltpu
from jax.experimental.pallas import tpu_sc as plsc

B, S = 1024, 50
N = B * S
D = 128
H = 256

NC, NS = 2, 16
NW = NC * NS
ROWS_PER_W = N // NW
CHUNK = 80
NCHUNK = ROWS_PER_W // CHUNK


def _sc_gather4(tabI, tabA, tabT, tabK, idxI, idxA, idxT, idxK):
    mesh = plsc.VectorSubcoreMesh(core_axis_name="c", subcore_axis_name="s")

    @functools.partial(
        pl.kernel,
        mesh=mesh,
        out_type=[jax.ShapeDtypeStruct((N, D), jnp.float32)] * 4,
        scratch_types=[
            pltpu.VMEM((NCHUNK, CHUNK), jnp.int32),
            pltpu.VMEM((NCHUNK, CHUNK), jnp.int32),
            pltpu.VMEM((NCHUNK, CHUNK), jnp.int32),
            pltpu.VMEM((NCHUNK, CHUNK), jnp.int32),
            pltpu.VMEM((CHUNK, D), jnp.float32),
            pltpu.VMEM((CHUNK, D), jnp.float32),
            pltpu.VMEM((CHUNK, D), jnp.float32),
            pltpu.VMEM((CHUNK, D), jnp.float32),
            pltpu.SemaphoreType.DMA,
            pltpu.SemaphoreType.DMA,
            pltpu.SemaphoreType.DMA,
            pltpu.SemaphoreType.DMA,
        ],
    )
    def k(tabI_h, tabA_h, tabT_h, tabK_h, idxI_h, idxA_h, idxT_h, idxK_h,
          outI_h, outA_h, outT_h, outK_h,
          idxI_v, idxA_v, idxT_v, idxK_v, bufI, bufA, bufT, bufK,
          semI, semA, semT, semK):
        wid = lax.axis_index("s") * NC + lax.axis_index("c")
        base = wid * ROWS_PER_W
        pltpu.sync_copy(idxI_h.at[wid], idxI_v)
        pltpu.sync_copy(idxA_h.at[wid], idxA_v)
        pltpu.sync_copy(idxT_h.at[wid], idxT_v)
        pltpu.sync_copy(idxK_h.at[wid], idxK_v)

        def body(j, carry):
            row0 = base + j * CHUNK
            cpI = pltpu.async_copy(tabI_h.at[idxI_v.at[j]], bufI, semI)
            cpA = pltpu.async_copy(tabA_h.at[idxA_v.at[j]], bufA, semA)
            cpT = pltpu.async_copy(tabT_h.at[idxT_v.at[j]], bufT, semT)
            cpK = pltpu.async_copy(tabK_h.at[idxK_v.at[j]], bufK, semK)
            cpI.wait()
            pltpu.sync_copy(bufI, outI_h.at[pl.ds(row0, CHUNK)])
            cpA.wait()
            pltpu.sync_copy(bufA, outA_h.at[pl.ds(row0, CHUNK)])
            cpT.wait()
            pltpu.sync_copy(bufT, outT_h.at[pl.ds(row0, CHUNK)])
            cpK.wait()
            pltpu.sync_copy(bufK, outK_h.at[pl.ds(row0, CHUNK)])
            return carry

        lax.fori_loop(0, NCHUNK, body, 0)

    return k(tabI, tabA, tabT, tabK, idxI, idxA, idxT, idxK)


R = 512
G = N // R


def _tc_body(gI, gA, gT, gK, xel, xua, xia, xta, W0, W1, W2, W3,
             cb, cg, cbeta, cW, ctb, ctg, ctbeta, out):
    eps = 1e-5
    acc = jnp.dot(gI[...], W0[...], preferred_element_type=jnp.float32)
    acc += jnp.dot(gA[...], W1[...], preferred_element_type=jnp.float32)
    acc += jnp.dot(gT[...], W2[...], preferred_element_type=jnp.float32)
    acc += jnp.dot(gK[...], W3[...], preferred_element_type=jnp.float32)
    x = acc + cb[...]
    mu = jnp.mean(x, axis=-1, keepdims=True)
    xc = x - mu
    var = jnp.mean(xc * xc, axis=-1, keepdims=True)
    cate = xc * lax.rsqrt(var + eps) * cg[...] + cbeta[...]

    y = (xel[0, 0, :][:, None] * cW[0:1, :]
         + xua[0, 0, :][:, None] * cW[1:2, :]
         + xia[0, 0, :][:, None] * cW[2:3, :]
         + xta[0, 0, :][:, None] * cW[3:4, :]
         + ctb[...])
    muy = jnp.mean(y, axis=-1, keepdims=True)
    yc = y - muy
    vary = jnp.mean(yc * yc, axis=-1, keepdims=True)
    cont = yc * lax.rsqrt(vary + eps) * ctg[...] + ctbeta[...]

    out[...] = jnp.concatenate([cate, cont], axis=-1)


def kernel(data_assessmentItemID, data_testId, data_KnowledgeTag, data_elapsed,
           data_user_acc, data_item_acc, data_tag_acc, data_answerCode,
           data_mask, data_interaction, emb_interaction, emb_assessmentItemID,
           emb_testId, emb_KnowledgeTag, comb_W, comb_b, comb_ln_g, comb_ln_b,
           cont_W, cont_b, cont_ln_g, cont_ln_b):
    idxI = data_interaction.reshape(NW, NCHUNK, CHUNK).astype(jnp.int32)
    idxA = data_assessmentItemID.reshape(NW, NCHUNK, CHUNK).astype(jnp.int32)
    idxT = data_testId.reshape(NW, NCHUNK, CHUNK).astype(jnp.int32)
    idxK = data_KnowledgeTag.reshape(NW, NCHUNK, CHUNK).astype(jnp.int32)

    gI, gA, gT, gK = _sc_gather4(emb_interaction, emb_assessmentItemID,
                                 emb_testId, emb_KnowledgeTag,
                                 idxI, idxA, idxT, idxK)

    xel = data_elapsed.reshape(G, 1, R)
    xua = data_user_acc.reshape(G, 1, R)
    xia = data_item_acc.reshape(G, 1, R)
    xta = data_tag_acc.reshape(G, 1, R)

    W0 = comb_W[0:D]
    W1 = comb_W[D:2 * D]
    W2 = comb_W[2 * D:3 * D]
    W3 = comb_W[3 * D:4 * D]
    cb = comb_b.reshape(1, H)
    cg = comb_ln_g.reshape(1, H)
    cbeta = comb_ln_b.reshape(1, H)
    cWp = jnp.zeros((8, H), jnp.float32).at[0:4].set(cont_W)
    ctb = cont_b.reshape(1, H)
    ctg = cont_ln_g.reshape(1, H)
    ctbeta = cont_ln_b.reshape(1, H)

    row_spec = pl.BlockSpec((R, D), lambda i: (i, 0))
    vec_spec = pl.BlockSpec((1, 1, R), lambda i: (i, 0, 0))
    full = lambda shape: pl.BlockSpec(shape, lambda i: tuple(0 for _ in shape))

    X = pl.pallas_call(
        _tc_body,
        grid=(G,),
        in_specs=[
            row_spec, row_spec, row_spec, row_spec,
            vec_spec, vec_spec, vec_spec, vec_spec,
            full((D, H)), full((D, H)), full((D, H)), full((D, H)),
            full((1, H)), full((1, H)), full((1, H)),
            full((8, H)), full((1, H)), full((1, H)), full((1, H)),
        ],
        out_specs=pl.BlockSpec((R, 2 * H), lambda i: (i, 0)),
        out_shape=jax.ShapeDtypeStruct((N, 2 * H), jnp.float32),
    )(gI, gA, gT, gK, xel, xua, xia, xta, W0, W1, W2, W3,
      cb, cg, cbeta, cWp, ctb, ctg, ctbeta)

    return X.reshape(B, S, 2 * H)


# speedup vs baseline: 2.9409x; 2.9409x over previous
import functools

import jax
import jax.numpy as jnp
from jax import lax
from jax.experimental import pallas as pl
from jax.experimental.pallas import tpu as pltpu
from jax.experimental.pallas import tpu_sc as plsc

B, S = 1024, 50
N = B * S
D = 128
H = 256

NC, NS = 2, 16
NW = NC * NS
ROWS_PER_W = N // NW
CHUNK = 80
NCHUNK = ROWS_PER_W // CHUNK


def _sc_gather3(tabA, tabT, tabK, idxA, idxT, idxK):
    mesh = plsc.VectorSubcoreMesh(core_axis_name="c", subcore_axis_name="s")

    @functools.partial(
        pl.kernel,
        mesh=mesh,
        out_type=[jax.ShapeDtypeStruct((N, D), jnp.float32)] * 3,
        scratch_types=(
            [pltpu.VMEM((NCHUNK, CHUNK), jnp.int32)] * 3
            + [pltpu.VMEM((CHUNK, D), jnp.float32)] * 6
            + [pltpu.SemaphoreType.DMA] * 12
        ),
    )
    def k(tabA_h, tabT_h, tabK_h, idxA_h, idxT_h, idxK_h,
          outA_h, outT_h, outK_h,
          idxA_v, idxT_v, idxK_v,
          bA0, bA1, bT0, bT1, bK0, bK1,
          gsA0, gsA1, gsT0, gsT1, gsK0, gsK1,
          wsA0, wsA1, wsT0, wsT1, wsK0, wsK1):
        wid = lax.axis_index("s") * NC + lax.axis_index("c")
        base = wid * ROWS_PER_W
        pltpu.sync_copy(idxA_h.at[wid], idxA_v)
        pltpu.sync_copy(idxT_h.at[wid], idxT_v)
        pltpu.sync_copy(idxK_h.at[wid], idxK_v)

        tabs = (
            (tabA_h, idxA_v, outA_h, bA0, bA1, gsA0, gsA1, wsA0, wsA1),
            (tabT_h, idxT_v, outT_h, bT0, bT1, gsT0, gsT1, wsT0, wsT1),
            (tabK_h, idxK_v, outK_h, bK0, bK1, gsK0, gsK1, wsK0, wsK1),
        )

        def g_cp(tab_h, idx_v, j, buf, sem):
            return pltpu.make_async_copy(tab_h.at[idx_v.at[j]], buf, sem)

        def w_cp(buf, out_h, row0, sem):
            return pltpu.make_async_copy(buf, out_h.at[pl.ds(row0, CHUNK)], sem)

        for (tab_h, idx_v, _o, b0, b1, g0, g1, _w0, _w1) in tabs:
            g_cp(tab_h, idx_v, 0, b0, g0).start()
            g_cp(tab_h, idx_v, 1, b1, g1).start()

        def body(m, carry):
            j0 = 2 * m
            j1 = j0 + 1
            r0 = base + j0 * CHUNK
            r1 = base + j1 * CHUNK
            for (tab_h, idx_v, out_h, b0, b1, g0, g1, w0, w1) in tabs:
                g_cp(tab_h, idx_v, j0, b0, g0).wait()
                w_cp(b0, out_h, r0, w0).start()
            for (tab_h, idx_v, out_h, b0, b1, g0, g1, w0, w1) in tabs:
                g_cp(tab_h, idx_v, j1, b1, g1).wait()
                w_cp(b1, out_h, r1, w1).start()
            for (tab_h, idx_v, out_h, b0, b1, g0, g1, w0, w1) in tabs:
                w_cp(b0, out_h, r0, w0).wait()
                g_cp(tab_h, idx_v, j0 + 2, b0, g0).start()
            for (tab_h, idx_v, out_h, b0, b1, g0, g1, w0, w1) in tabs:
                w_cp(b1, out_h, r1, w1).wait()
                g_cp(tab_h, idx_v, j1 + 2, b1, g1).start()
            return carry

        lax.fori_loop(0, NCHUNK // 2 - 1, body, 0)

        jl0 = NCHUNK - 2
        jl1 = NCHUNK - 1
        for (tab_h, idx_v, out_h, b0, b1, g0, g1, w0, w1) in tabs:
            g_cp(tab_h, idx_v, jl0, b0, g0).wait()
            w_cp(b0, out_h, base + jl0 * CHUNK, w0).start()
            g_cp(tab_h, idx_v, jl1, b1, g1).wait()
            w_cp(b1, out_h, base + jl1 * CHUNK, w1).start()
        for (tab_h, idx_v, out_h, b0, b1, g0, g1, w0, w1) in tabs:
            w_cp(b0, out_h, base + jl0 * CHUNK, w0).wait()
            w_cp(b1, out_h, base + jl1 * CHUNK, w1).wait()

    return k(tabA, tabT, tabK, idxA, idxT, idxK)


R = 512
G = N // R


def _tc_body(gA, gT, gK, ii, xel, xua, xia, xta, embI, W0, W1, W2, W3,
             cb, cg, cbeta, cW, ctb, ctg, ctbeta, out):
    eps = 1e-5
    acc = jnp.dot(gA[...], W1[...], preferred_element_type=jnp.float32)
    acc += jnp.dot(gT[...], W2[...], preferred_element_type=jnp.float32)
    acc += jnp.dot(gK[...], W3[...], preferred_element_type=jnp.float32)
    P0 = jnp.dot(embI[...], W0[...], preferred_element_type=jnp.float32)
    iif = ii[0, 0, :].astype(jnp.float32)[:, None]
    s0 = jnp.maximum(0.0, 1.0 - jnp.abs(iif))
    s1 = jnp.maximum(0.0, 1.0 - jnp.abs(iif - 1.0))
    s2 = jnp.maximum(0.0, 1.0 - jnp.abs(iif - 2.0))
    p = s0 * P0[0:1, :] + s1 * P0[1:2, :] + s2 * P0[2:3, :]
    x = acc + p + cb[...]
    mu = jnp.mean(x, axis=-1, keepdims=True)
    xc = x - mu
    var = jnp.mean(xc * xc, axis=-1, keepdims=True)
    cate = xc * lax.rsqrt(var + eps) * cg[...] + cbeta[...]

    y = (xel[0, 0, :][:, None] * cW[0:1, :]
         + xua[0, 0, :][:, None] * cW[1:2, :]
         + xia[0, 0, :][:, None] * cW[2:3, :]
         + xta[0, 0, :][:, None] * cW[3:4, :]
         + ctb[...])
    muy = jnp.mean(y, axis=-1, keepdims=True)
    yc = y - muy
    vary = jnp.mean(yc * yc, axis=-1, keepdims=True)
    cont = yc * lax.rsqrt(vary + eps) * ctg[...] + ctbeta[...]

    out[...] = jnp.concatenate([cate, cont], axis=-1)


def kernel(data_assessmentItemID, data_testId, data_KnowledgeTag, data_elapsed,
           data_user_acc, data_item_acc, data_tag_acc, data_answerCode,
           data_mask, data_interaction, emb_interaction, emb_assessmentItemID,
           emb_testId, emb_KnowledgeTag, comb_W, comb_b, comb_ln_g, comb_ln_b,
           cont_W, cont_b, cont_ln_g, cont_ln_b):
    idxA = data_assessmentItemID.reshape(NW, NCHUNK, CHUNK).astype(jnp.int32)
    idxT = data_testId.reshape(NW, NCHUNK, CHUNK).astype(jnp.int32)
    idxK = data_KnowledgeTag.reshape(NW, NCHUNK, CHUNK).astype(jnp.int32)

    gA, gT, gK = _sc_gather3(emb_assessmentItemID, emb_testId,
                             emb_KnowledgeTag, idxA, idxT, idxK)

    ii = data_interaction.reshape(G, 1, R).astype(jnp.int32)
    xel = data_elapsed.reshape(G, 1, R)
    xua = data_user_acc.reshape(G, 1, R)
    xia = data_item_acc.reshape(G, 1, R)
    xta = data_tag_acc.reshape(G, 1, R)

    embI = jnp.zeros((8, D), jnp.float32).at[0:3].set(emb_interaction)
    W0 = comb_W[0:D]
    W1 = comb_W[D:2 * D]
    W2 = comb_W[2 * D:3 * D]
    W3 = comb_W[3 * D:4 * D]
    cb = comb_b.reshape(1, H)
    cg = comb_ln_g.reshape(1, H)
    cbeta = comb_ln_b.reshape(1, H)
    cWp = jnp.zeros((8, H), jnp.float32).at[0:4].set(cont_W)
    ctb = cont_b.reshape(1, H)
    ctg = cont_ln_g.reshape(1, H)
    ctbeta = cont_ln_b.reshape(1, H)

    row_spec = pl.BlockSpec((R, D), lambda i: (i, 0))
    vec_spec = pl.BlockSpec((1, 1, R), lambda i: (i, 0, 0))
    full = lambda shape: pl.BlockSpec(shape, lambda i: tuple(0 for _ in shape))

    X = pl.pallas_call(
        _tc_body,
        grid=(G,),
        in_specs=[
            row_spec, row_spec, row_spec,
            vec_spec, vec_spec, vec_spec, vec_spec, vec_spec,
            full((8, D)), full((D, H)), full((D, H)), full((D, H)),
            full((D, H)), full((1, H)), full((1, H)), full((1, H)),
            full((8, H)), full((1, H)), full((1, H)), full((1, H)),
        ],
        out_specs=pl.BlockSpec((R, 2 * H), lambda i: (i, 0)),
        out_shape=jax.ShapeDtypeStruct((N, 2 * H), jnp.float32),
    )(gA, gT, gK, ii, xel, xua, xia, xta, embI, W0, W1, W2, W3,
      cb, cg, cbeta, cWp, ctb, ctg, ctbeta)

    return X.reshape(B, S, 2 * H)


# speedup vs baseline: 5.0805x; 1.7275x over previous
import functools

import jax
import jax.numpy as jnp
from jax import lax
from jax.experimental import pallas as pl
from jax.experimental.pallas import tpu as pltpu
from jax.experimental.pallas import tpu_sc as plsc

B, S = 1024, 50
N = B * S
D = 128
H = 256

NC, NS = 2, 16
NW = NC * NS
ROWS_PER_W = N // NW
CHUNK = 80
NCHUNK = ROWS_PER_W // CHUNK


def _sc_gather3(tabA, tabT, tabK, idxA, idxT, idxK):
    mesh = plsc.VectorSubcoreMesh(core_axis_name="c", subcore_axis_name="s")

    @functools.partial(
        pl.kernel,
        mesh=mesh,
        out_type=[jax.ShapeDtypeStruct((N, D), jnp.float32)] * 3,
        scratch_types=(
            [pltpu.VMEM((NCHUNK, CHUNK), jnp.int32)] * 3
            + [pltpu.VMEM((CHUNK, D), jnp.float32)] * 6
            + [pltpu.SemaphoreType.DMA] * 12
        ),
    )
    def k(tabA_h, tabT_h, tabK_h, idxA_h, idxT_h, idxK_h,
          outA_h, outT_h, outK_h,
          idxA_v, idxT_v, idxK_v,
          bA0, bA1, bT0, bT1, bK0, bK1,
          gsA0, gsA1, gsT0, gsT1, gsK0, gsK1,
          wsA0, wsA1, wsT0, wsT1, wsK0, wsK1):
        wid = lax.axis_index("s") * NC + lax.axis_index("c")
        base = wid * ROWS_PER_W
        pltpu.sync_copy(idxA_h.at[wid], idxA_v)
        pltpu.sync_copy(idxT_h.at[wid], idxT_v)
        pltpu.sync_copy(idxK_h.at[wid], idxK_v)

        tabs = (
            (tabA_h, idxA_v, outA_h, bA0, bA1, gsA0, gsA1, wsA0, wsA1),
            (tabT_h, idxT_v, outT_h, bT0, bT1, gsT0, gsT1, wsT0, wsT1),
            (tabK_h, idxK_v, outK_h, bK0, bK1, gsK0, gsK1, wsK0, wsK1),
        )

        def g_cp(tab_h, idx_v, j, buf, sem):
            return pltpu.make_async_copy(tab_h.at[idx_v.at[j]], buf, sem)

        def w_cp(buf, out_h, row0, sem):
            return pltpu.make_async_copy(buf, out_h.at[pl.ds(row0, CHUNK)], sem)

        for (tab_h, idx_v, _o, b0, b1, g0, g1, _w0, _w1) in tabs:
            g_cp(tab_h, idx_v, 0, b0, g0).start()
            g_cp(tab_h, idx_v, 1, b1, g1).start()

        def body(m, carry):
            j0 = 2 * m
            j1 = j0 + 1
            r0 = base + j0 * CHUNK
            r1 = base + j1 * CHUNK
            for (tab_h, idx_v, out_h, b0, b1, g0, g1, w0, w1) in tabs:
                g_cp(tab_h, idx_v, j0, b0, g0).wait()
                w_cp(b0, out_h, r0, w0).start()
            for (tab_h, idx_v, out_h, b0, b1, g0, g1, w0, w1) in tabs:
                g_cp(tab_h, idx_v, j1, b1, g1).wait()
                w_cp(b1, out_h, r1, w1).start()
            for (tab_h, idx_v, out_h, b0, b1, g0, g1, w0, w1) in tabs:
                w_cp(b0, out_h, r0, w0).wait()
                g_cp(tab_h, idx_v, j0 + 2, b0, g0).start()
            for (tab_h, idx_v, out_h, b0, b1, g0, g1, w0, w1) in tabs:
                w_cp(b1, out_h, r1, w1).wait()
                g_cp(tab_h, idx_v, j1 + 2, b1, g1).start()
            return carry

        lax.fori_loop(0, NCHUNK // 2 - 1, body, 0)

        jl0 = NCHUNK - 2
        jl1 = NCHUNK - 1
        for (tab_h, idx_v, out_h, b0, b1, g0, g1, w0, w1) in tabs:
            g_cp(tab_h, idx_v, jl0, b0, g0).wait()
            w_cp(b0, out_h, base + jl0 * CHUNK, w0).start()
            g_cp(tab_h, idx_v, jl1, b1, g1).wait()
            w_cp(b1, out_h, base + jl1 * CHUNK, w1).start()
        for (tab_h, idx_v, out_h, b0, b1, g0, g1, w0, w1) in tabs:
            w_cp(b0, out_h, base + jl0 * CHUNK, w0).wait()
            w_cp(b1, out_h, base + jl1 * CHUNK, w1).wait()

    return k(tabA, tabT, tabK, idxA, idxT, idxK)


R = 512
G = N // R


def _tc_body(gA, gT, gK, ii, xel, xua, xia, xta, embI, W0, W1, W2, W3,
             cb, cg, cbeta, cW, ctb, ctg, ctbeta, out):
    eps = 1e-5
    acc = jnp.dot(gA[...], W1[...], preferred_element_type=jnp.float32)
    acc += jnp.dot(gT[...], W2[...], preferred_element_type=jnp.float32)
    acc += jnp.dot(gK[...], W3[...], preferred_element_type=jnp.float32)
    P0 = jnp.dot(embI[...], W0[...], preferred_element_type=jnp.float32)
    iif = ii[0, 0, :].astype(jnp.float32)[:, None]
    s0 = jnp.maximum(0.0, 1.0 - jnp.abs(iif))
    s1 = jnp.maximum(0.0, 1.0 - jnp.abs(iif - 1.0))
    s2 = jnp.maximum(0.0, 1.0 - jnp.abs(iif - 2.0))
    p = s0 * P0[0:1, :] + s1 * P0[1:2, :] + s2 * P0[2:3, :]
    x = acc + p + cb[...]
    mu = jnp.mean(x, axis=-1, keepdims=True)
    xc = x - mu
    var = jnp.mean(xc * xc, axis=-1, keepdims=True)
    cate = xc * lax.rsqrt(var + eps) * cg[...] + cbeta[...]

    y = (xel[0, 0, :][:, None] * cW[0:1, :]
         + xua[0, 0, :][:, None] * cW[1:2, :]
         + xia[0, 0, :][:, None] * cW[2:3, :]
         + xta[0, 0, :][:, None] * cW[3:4, :]
         + ctb[...])
    muy = jnp.mean(y, axis=-1, keepdims=True)
    yc = y - muy
    vary = jnp.mean(yc * yc, axis=-1, keepdims=True)
    cont = yc * lax.rsqrt(vary + eps) * ctg[...] + ctbeta[...]

    out[...] = jnp.concatenate([cate, cont], axis=-1)


def kernel(data_assessmentItemID, data_testId, data_KnowledgeTag, data_elapsed,
           data_user_acc, data_item_acc, data_tag_acc, data_answerCode,
           data_mask, data_interaction, emb_interaction, emb_assessmentItemID,
           emb_testId, emb_KnowledgeTag, comb_W, comb_b, comb_ln_g, comb_ln_b,
           cont_W, cont_b, cont_ln_g, cont_ln_b):
    idxA = data_assessmentItemID.T.reshape(NW, NCHUNK, CHUNK).astype(jnp.int32)
    idxT = data_testId.T.reshape(NW, NCHUNK, CHUNK).astype(jnp.int32)
    idxK = data_KnowledgeTag.T.reshape(NW, NCHUNK, CHUNK).astype(jnp.int32)

    gA, gT, gK = _sc_gather3(emb_assessmentItemID, emb_testId,
                             emb_KnowledgeTag, idxA, idxT, idxK)

    ii = data_interaction.T.reshape(G, 1, R).astype(jnp.int32)
    xel = data_elapsed.T.reshape(G, 1, R)
    xua = data_user_acc.T.reshape(G, 1, R)
    xia = data_item_acc.T.reshape(G, 1, R)
    xta = data_tag_acc.T.reshape(G, 1, R)

    embI = jnp.zeros((8, D), jnp.float32).at[0:3].set(emb_interaction)
    W0 = comb_W[0:D]
    W1 = comb_W[D:2 * D]
    W2 = comb_W[2 * D:3 * D]
    W3 = comb_W[3 * D:4 * D]
    cb = comb_b.reshape(1, H)
    cg = comb_ln_g.reshape(1, H)
    cbeta = comb_ln_b.reshape(1, H)
    cWp = jnp.zeros((8, H), jnp.float32).at[0:4].set(cont_W)
    ctb = cont_b.reshape(1, H)
    ctg = cont_ln_g.reshape(1, H)
    ctbeta = cont_ln_b.reshape(1, H)

    row_spec = pl.BlockSpec((R, D), lambda i: (i, 0))
    vec_spec = pl.BlockSpec((1, 1, R), lambda i: (i, 0, 0))
    full = lambda shape: pl.BlockSpec(shape, lambda i: tuple(0 for _ in shape))

    X = pl.pallas_call(
        _tc_body,
        grid=(G,),
        in_specs=[
            row_spec, row_spec, row_spec,
            vec_spec, vec_spec, vec_spec, vec_spec, vec_spec,
            full((8, D)), full((D, H)), full((D, H)), full((D, H)),
            full((D, H)), full((1, H)), full((1, H)), full((1, H)),
            full((8, H)), full((1, H)), full((1, H)), full((1, H)),
        ],
        out_specs=pl.BlockSpec((R, 2 * H), lambda i: (i, 0)),
        out_shape=jax.ShapeDtypeStruct((N, 2 * H), jnp.float32),
    )(gA, gT, gK, ii, xel, xua, xia, xta, embI, W0, W1, W2, W3,
      cb, cg, cbeta, cWp, ctb, ctg, ctbeta)

    return X.reshape(S, B, 2 * H).transpose(1, 0, 2)


# speedup vs baseline: 6.9900x; 1.3758x over previous
import functools

import jax
import jax.numpy as jnp
from jax import lax
from jax.experimental import pallas as pl
from jax.experimental.pallas import tpu as pltpu
from jax.experimental.pallas import tpu_sc as plsc

B, S = 1024, 50
N = B * S
D = 128
H = 256

NC, NS = 2, 16
NW = NC * NS
ROWS_PER_W = N // NW
CHUNK = 80
NCHUNK = ROWS_PER_W // CHUNK


def _sc_gather3(tabA, tabT, tabK, idxA, idxT, idxK):
    mesh = plsc.VectorSubcoreMesh(core_axis_name="c", subcore_axis_name="s")

    @functools.partial(
        pl.kernel,
        mesh=mesh,
        out_type=[jax.ShapeDtypeStruct((N, D), jnp.float32)] * 3,
        scratch_types=(
            [pltpu.VMEM((NCHUNK, CHUNK), jnp.int32)] * 3
            + [pltpu.VMEM((CHUNK, D), jnp.float32)] * 6
            + [pltpu.SemaphoreType.DMA] * 12
        ),
    )
    def k(tabA_h, tabT_h, tabK_h, idxA_h, idxT_h, idxK_h,
          outA_h, outT_h, outK_h,
          idxA_v, idxT_v, idxK_v,
          bA0, bA1, bT0, bT1, bK0, bK1,
          gsA0, gsA1, gsT0, gsT1, gsK0, gsK1,
          wsA0, wsA1, wsT0, wsT1, wsK0, wsK1):
        wid = lax.axis_index("s") * NC + lax.axis_index("c")
        base = wid * ROWS_PER_W
        pltpu.sync_copy(idxA_h.at[wid], idxA_v)
        pltpu.sync_copy(idxT_h.at[wid], idxT_v)
        pltpu.sync_copy(idxK_h.at[wid], idxK_v)

        tabs = (
            (tabA_h, idxA_v, outA_h, bA0, bA1, gsA0, gsA1, wsA0, wsA1),
            (tabT_h, idxT_v, outT_h, bT0, bT1, gsT0, gsT1, wsT0, wsT1),
            (tabK_h, idxK_v, outK_h, bK0, bK1, gsK0, gsK1, wsK0, wsK1),
        )

        def g_cp(tab_h, idx_v, j, buf, sem):
            return pltpu.make_async_copy(tab_h.at[idx_v.at[j]], buf, sem)

        def w_cp(buf, out_h, row0, sem):
            return pltpu.make_async_copy(buf, out_h.at[pl.ds(row0, CHUNK)], sem)

        for (tab_h, idx_v, _o, b0, b1, g0, g1, _w0, _w1) in tabs:
            g_cp(tab_h, idx_v, 0, b0, g0).start()
            g_cp(tab_h, idx_v, 1, b1, g1).start()

        def body(m, carry):
            j0 = 2 * m
            j1 = j0 + 1
            r0 = base + j0 * CHUNK
            r1 = base + j1 * CHUNK
            for (tab_h, idx_v, out_h, b0, b1, g0, g1, w0, w1) in tabs:
                g_cp(tab_h, idx_v, j0, b0, g0).wait()
                w_cp(b0, out_h, r0, w0).start()
            for (tab_h, idx_v, out_h, b0, b1, g0, g1, w0, w1) in tabs:
                g_cp(tab_h, idx_v, j1, b1, g1).wait()
                w_cp(b1, out_h, r1, w1).start()
            for (tab_h, idx_v, out_h, b0, b1, g0, g1, w0, w1) in tabs:
                w_cp(b0, out_h, r0, w0).wait()
                g_cp(tab_h, idx_v, j0 + 2, b0, g0).start()
            for (tab_h, idx_v, out_h, b0, b1, g0, g1, w0, w1) in tabs:
                w_cp(b1, out_h, r1, w1).wait()
                g_cp(tab_h, idx_v, j1 + 2, b1, g1).start()
            return carry

        lax.fori_loop(0, NCHUNK // 2 - 1, body, 0)

        jl0 = NCHUNK - 2
        jl1 = NCHUNK - 1
        for (tab_h, idx_v, out_h, b0, b1, g0, g1, w0, w1) in tabs:
            g_cp(tab_h, idx_v, jl0, b0, g0).wait()
            w_cp(b0, out_h, base + jl0 * CHUNK, w0).start()
            g_cp(tab_h, idx_v, jl1, b1, g1).wait()
            w_cp(b1, out_h, base + jl1 * CHUNK, w1).start()
        for (tab_h, idx_v, out_h, b0, b1, g0, g1, w0, w1) in tabs:
            w_cp(b0, out_h, base + jl0 * CHUNK, w0).wait()
            w_cp(b1, out_h, base + jl1 * CHUNK, w1).wait()

    return k(tabA, tabT, tabK, idxA, idxT, idxK)


R = 2048
G = N // R


_DN_T = (((0,), (0,)), ((), ()))


def _tc_body(gA, gT, gK, ii, xel, xua, xia, xta, embI, W0, W1, W2, W3,
             cb, cg, cbeta, cW, ctb, ctg, ctbeta, out):
    eps = 1e-5
    acc = jnp.dot(gA[...], W1[...], preferred_element_type=jnp.float32)
    acc += jnp.dot(gT[...], W2[...], preferred_element_type=jnp.float32)
    acc += jnp.dot(gK[...], W3[...], preferred_element_type=jnp.float32)
    P0 = jnp.dot(embI[...], W0[...], preferred_element_type=jnp.float32)
    iif = ii[0].astype(jnp.float32)
    s0 = jnp.maximum(0.0, 1.0 - jnp.abs(iif))
    s1 = jnp.maximum(0.0, 1.0 - jnp.abs(iif - 1.0))
    s2 = jnp.maximum(0.0, 1.0 - jnp.abs(iif - 2.0))
    oh = jnp.concatenate(
        [s0, s1, s2, jnp.zeros((5, s0.shape[1]), jnp.float32)], axis=0)
    p = lax.dot_general(oh, P0, _DN_T, preferred_element_type=jnp.float32)
    x = acc + p + cb[...]
    mu = jnp.mean(x, axis=-1, keepdims=True)
    xc = x - mu
    var = jnp.mean(xc * xc, axis=-1, keepdims=True)
    cate = xc * lax.rsqrt(var + eps) * cg[...] + cbeta[...]

    xq = jnp.concatenate(
        [xel[0], xua[0], xia[0], xta[0],
         jnp.zeros((4, xel.shape[2]), jnp.float32)], axis=0)
    y = lax.dot_general(xq, cW[...], _DN_T,
                        preferred_element_type=jnp.float32) + ctb[...]
    muy = jnp.mean(y, axis=-1, keepdims=True)
    yc = y - muy
    vary = jnp.mean(yc * yc, axis=-1, keepdims=True)
    cont = yc * lax.rsqrt(vary + eps) * ctg[...] + ctbeta[...]

    out[...] = jnp.concatenate([cate, cont], axis=-1)


def kernel(data_assessmentItemID, data_testId, data_KnowledgeTag, data_elapsed,
           data_user_acc, data_item_acc, data_tag_acc, data_answerCode,
           data_mask, data_interaction, emb_interaction, emb_assessmentItemID,
           emb_testId, emb_KnowledgeTag, comb_W, comb_b, comb_ln_g, comb_ln_b,
           cont_W, cont_b, cont_ln_g, cont_ln_b):
    idxA = data_assessmentItemID.T.reshape(NW, NCHUNK, CHUNK).astype(jnp.int32)
    idxT = data_testId.T.reshape(NW, NCHUNK, CHUNK).astype(jnp.int32)
    idxK = data_KnowledgeTag.T.reshape(NW, NCHUNK, CHUNK).astype(jnp.int32)

    gA, gT, gK = _sc_gather3(emb_assessmentItemID, emb_testId,
                             emb_KnowledgeTag, idxA, idxT, idxK)

    ii = data_interaction.T.reshape(G, 1, R).astype(jnp.int32)
    xel = data_elapsed.T.reshape(G, 1, R)
    xua = data_user_acc.T.reshape(G, 1, R)
    xia = data_item_acc.T.reshape(G, 1, R)
    xta = data_tag_acc.T.reshape(G, 1, R)

    embI = jnp.zeros((8, D), jnp.float32).at[0:3].set(emb_interaction)
    W0 = comb_W[0:D]
    W1 = comb_W[D:2 * D]
    W2 = comb_W[2 * D:3 * D]
    W3 = comb_W[3 * D:4 * D]
    cb = comb_b.reshape(1, H)
    cg = comb_ln_g.reshape(1, H)
    cbeta = comb_ln_b.reshape(1, H)
    cWp = jnp.zeros((8, H), jnp.float32).at[0:4].set(cont_W)
    ctb = cont_b.reshape(1, H)
    ctg = cont_ln_g.reshape(1, H)
    ctbeta = cont_ln_b.reshape(1, H)

    row_spec = pl.BlockSpec((R, D), lambda i: (i, 0))
    vec_spec = pl.BlockSpec((1, 1, R), lambda i: (i, 0, 0))
    full = lambda shape: pl.BlockSpec(shape, lambda i: tuple(0 for _ in shape))

    X = pl.pallas_call(
        _tc_body,
        grid=(G,),
        in_specs=[
            row_spec, row_spec, row_spec,
            vec_spec, vec_spec, vec_spec, vec_spec, vec_spec,
            full((8, D)), full((D, H)), full((D, H)), full((D, H)),
            full((D, H)), full((1, H)), full((1, H)), full((1, H)),
            full((8, H)), full((1, H)), full((1, H)), full((1, H)),
        ],
        out_specs=pl.BlockSpec((R, 2 * H), lambda i: (i, 0)),
        out_shape=jax.ShapeDtypeStruct((N, 2 * H), jnp.float32),
    )(gA, gT, gK, ii, xel, xua, xia, xta, embI, W0, W1, W2, W3,
      cb, cg, cbeta, cWp, ctb, ctg, ctbeta)

    return X.reshape(S, B, 2 * H).transpose(1, 0, 2)
